# baseline (device time: 66703 ns/iter reference)
import jax
import jax.numpy as jnp
from jax import lax
from jax.experimental import pallas as pl
from jax.experimental.pallas import tpu as pltpu

N_DEV = 16
M = 1024
N = 1024
QROWS = 256
PROWS = 64
STRIP = 256
NSEM = 18
CHS = ("AR", "AL", "BR", "BL")


def kernel(A, B):
    def body(a_ref, b_ref, out_ref,
             cAR1, cAL1, cBR1, cBL1, cAR2, cAL2, cBR2, cBL2,
             sAR, rAR, sAL, rAL, sBR, rBR, sBL, rBL):
        my = lax.axis_index("i")
        z4 = lax.div(my, 4)
        q4 = lax.rem(my, 4)

        def m4(v):
            return lax.rem(v + 8, 4)

        pr = z4 * 4 + m4(q4 + 1)
        plq = z4 * 4 + m4(q4 - 1)
        zr = m4(z4 + 1) * 4 + q4
        zl = m4(z4 - 1) * 4 + q4

        cols = {ch: pl.ds(k * STRIP, STRIP) for k, ch in enumerate(CHS)}
        sems = {"AR": (sAR, rAR), "AL": (sAL, rAL),
                "BR": (sBR, rBR), "BL": (sBL, rBL)}
        comm1 = {"AR": cAR1, "AL": cAL1, "BR": cBR1, "BL": cBL1}
        comm2 = {"AR": cAR2, "AL": cAL2, "BR": cBR2, "BL": cBL2}
        geom = {
            "AR": (q4, pr, z4, zr, +1),
            "AL": (q4, plq, z4, zl, -1),
            "BR": (z4, zr, q4, pr, +1),
            "BL": (z4, zl, q4, plq, -1),
        }
        qown = {k: m4(geom[k][0] + geom[k][4]) for k in geom}
        p1idx = {ch: [m4(geom[ch][2] - geom[ch][4] * j) for j in (0, 1, 2, -1)]
                 for ch in CHS}
        P1SEM = (2, 12, 13, 14)
        p4idx = {ch: [m4(geom[ch][2] + geom[ch][4] * j) for j in (1, 0, -1, -2)]
                 for ch in CHS}
        P4SEM = (9, 15, 16, 17)

        def qrows(qi):
            return pl.ds(qi * QROWS, QROWS)

        def prow(qi, pi):
            return pl.ds(qi * QROWS + pi * PROWS, PROWS)

        all_rdmas = []

        def mkrdma(src, dst, ch, si, tgt):
            snd, rcv = sems[ch]
            r = pltpu.make_async_remote_copy(
                src_ref=src, dst_ref=dst,
                send_sem=snd.at[si], recv_sem=rcv.at[si],
                device_id=(tgt,), device_id_type=pl.DeviceIdType.MESH)
            all_rdmas.append(r)
            r.start()
            return r

        def start(ch, s):
            v, tgt, v2, tgt2, d = geom[ch]
            if s < 2:
                src = out_ref.at[qrows(m4(v - d * s)), cols[ch]]
                return mkrdma(src, comm1[ch].at[s], ch, s, tgt)
            if s < 6:
                src = out_ref.at[prow(qown[ch], m4(v2 - d * (s - 3))), cols[ch]]
                return mkrdma(src, comm2[ch].at[s - 3], ch, s, tgt2)
            if s < 9:
                src = out_ref.at[prow(qown[ch], m4(v2 + d * (7 - s))), cols[ch]]
                return mkrdma(src, src, ch, s, tgt2)
            src = out_ref.at[qrows(m4(v + d * (10 - s))), cols[ch]]
            return mkrdma(src, src, ch, s, tgt)

        def start_p1(ch, k):
            v, tgt, _, _, d = geom[ch]
            p = p1idx[ch][k]
            src = out_ref.at[prow(m4(v - 2 * d), p), cols[ch]]
            dst = comm1[ch].at[2, pl.ds(p * PROWS, PROWS), :]
            return mkrdma(src, dst, ch, P1SEM[k], tgt)

        def start_p4(ch, k):
            _, tgt, _, _, _ = geom[ch]
            src = out_ref.at[prow(qown[ch], p4idx[ch][k]), cols[ch]]
            return mkrdma(src, src, ch, P4SEM[k], tgt)

        def acc_p1(ch, k):
            p = p1idx[ch][k]
            out_ref[prow(qown[ch], p), cols[ch]] = (
                out_ref[prow(qown[ch], p), cols[ch]]
                + comm1[ch][2, pl.ds(p * PROWS, PROWS), :])

        def accumulate(ch, s):
            v, _, v2, _, d = geom[ch]
            if s < 3:
                rq = m4(v - d * (s + 1))
                out_ref[qrows(rq), cols[ch]] = (
                    out_ref[qrows(rq), cols[ch]] + comm1[ch][s])
            else:
                rp = m4(v2 - d * (s - 2))
                out_ref[prow(qown[ch], rp), cols[ch]] = (
                    out_ref[prow(qown[ch], rp), cols[ch]] + comm2[ch][s - 3])

        def compute_quarter(idx):
            out_ref[qrows(idx), :] = jnp.dot(
                a_ref[qrows(idx), :], b_ref[:, :],
                preferred_element_type=jnp.float32)

        barrier_sem = pltpu.get_barrier_semaphore()
        for nbr in (pr, plq, zr, zl):
            pl.semaphore_signal(barrier_sem, inc=1, device_id=(nbr,),
                                device_id_type=pl.DeviceIdType.MESH)
        pl.semaphore_wait(barrier_sem, 4)

        compute_quarter(q4)
        live = {}
        live["AR"] = start("AR", 0)
        live["AL"] = start("AL", 0)
        pl.when(z4 != q4)(lambda: compute_quarter(z4))
        live["BR"] = start("BR", 0)
        live["BL"] = start("BL", 0)
        for j in range(4):
            pl.when((j != q4) & (j != z4))(lambda j=j: compute_quarter(j))

        p1d = {}
        for s in (0, 1):
            for ch in CHS:
                live[ch].wait_recv()
                accumulate(ch, s)
                if s == 0:
                    live[ch] = start(ch, 1)
                else:
                    p1d[ch] = [start_p1(ch, k) for k in range(4)]

        for ch in CHS:
            p1d[ch][0].wait_recv()
            acc_p1(ch, 0)
            live[ch] = start(ch, 3)
        for t in range(3):
            for ch in CHS:
                p1d[ch][t + 1].wait_recv()
                acc_p1(ch, t + 1)
                live[ch].wait_recv()
                accumulate(ch, 3 + t)
                if t < 2:
                    live[ch] = start(ch, 4 + t)

        p4d = {}
        for ch in CHS:
            live[ch] = start(ch, 6)
            p4d[ch] = [start_p4(ch, 0)]
        for t in range(2):
            for ch in CHS:
                live[ch].wait_recv()
                p4d[ch].append(start_p4(ch, t + 1))
                live[ch] = start(ch, 7 + t)
        for ch in CHS:
            live[ch].wait_recv()
            p4d[ch].append(start_p4(ch, 3))

        for ch in CHS:
            for dsc in p4d[ch]:
                dsc.wait_recv()
            live[ch] = start(ch, 10)
        for ch in CHS:
            live[ch].wait_recv()
            live[ch] = start(ch, 11)
        for ch in CHS:
            live[ch].wait_recv()

        for r in all_rdmas:
            r.wait_send()

    return pl.pallas_call(
        body,
        out_shape=jax.ShapeDtypeStruct((M, N), jnp.float32),
        in_specs=[
            pl.BlockSpec(memory_space=pltpu.VMEM),
            pl.BlockSpec(memory_space=pltpu.VMEM),
        ],
        out_specs=pl.BlockSpec(memory_space=pltpu.VMEM),
        scratch_shapes=(
            [pltpu.VMEM((3, QROWS, STRIP), jnp.float32) for _ in range(4)]
            + [pltpu.VMEM((3, PROWS, STRIP), jnp.float32) for _ in range(4)]
            + [pltpu.SemaphoreType.DMA((NSEM,)) for _ in range(8)]
        ),
        compiler_params=pltpu.CompilerParams(collective_id=0),
    )(A, B)


# device time: 66623 ns/iter; 1.0012x vs baseline; 1.0012x over previous
import jax
import jax.numpy as jnp
from jax import lax
from jax.experimental import pallas as pl
from jax.experimental.pallas import tpu as pltpu

N_DEV = 16
M = 1024
N = 1024
QROWS = 256
PROWS = 64
STRIP = 256
NSEM = 18
CHS = ("AR", "AL", "BR", "BL")


def kernel(A, B):
    def body(a_ref, b_ref, out_ref,
             cAR1, cAL1, cBR1, cBL1, cAR2, cAL2, cBR2, cBL2,
             sAR, rAR, sAL, rAL, sBR, rBR, sBL, rBL):
        my = lax.axis_index("i")
        z4 = lax.div(my, 4)
        q4 = lax.rem(my, 4)

        def m4(v):
            return lax.rem(v + 8, 4)

        pr = z4 * 4 + m4(q4 + 1)
        plq = z4 * 4 + m4(q4 - 1)
        zr = m4(z4 + 1) * 4 + q4
        zl = m4(z4 - 1) * 4 + q4

        cols = {ch: pl.ds(k * STRIP, STRIP) for k, ch in enumerate(CHS)}
        sems = {"AR": (sAR, rAR), "AL": (sAL, rAL),
                "BR": (sBR, rBR), "BL": (sBL, rBL)}
        comm1 = {"AR": cAR1, "AL": cAL1, "BR": cBR1, "BL": cBL1}
        comm2 = {"AR": cAR2, "AL": cAL2, "BR": cBR2, "BL": cBL2}
        geom = {
            "AR": (q4, pr, z4, zr, +1),
            "AL": (q4, plq, z4, zl, -1),
            "BR": (z4, zr, q4, pr, +1),
            "BL": (z4, zl, q4, plq, -1),
        }
        qown = {k: m4(geom[k][0] + geom[k][4]) for k in geom}
        p1idx = {ch: [m4(geom[ch][2] - geom[ch][4] * j) for j in (0, 1, 2, -1)]
                 for ch in CHS}
        P1SEM = (2, 12, 13, 14)
        p4idx = {ch: [m4(geom[ch][2] + geom[ch][4] * j) for j in (1, 0, -1, -2)]
                 for ch in CHS}
        P4SEM = (9, 15, 16, 17)

        def qrows(qi):
            return pl.ds(qi * QROWS, QROWS)

        def prow(qi, pi):
            return pl.ds(qi * QROWS + pi * PROWS, PROWS)

        all_rdmas = []

        def mkrdma(src, dst, ch, si, tgt):
            snd, rcv = sems[ch]
            r = pltpu.make_async_remote_copy(
                src_ref=src, dst_ref=dst,
                send_sem=snd.at[si], recv_sem=rcv.at[si],
                device_id=(tgt,), device_id_type=pl.DeviceIdType.MESH)
            all_rdmas.append(r)
            r.start()
            return r

        def start(ch, s):
            v, tgt, v2, tgt2, d = geom[ch]
            if s < 2:
                src = out_ref.at[qrows(m4(v - d * s)), cols[ch]]
                return mkrdma(src, comm1[ch].at[s], ch, s, tgt)
            if s < 6:
                src = out_ref.at[prow(qown[ch], m4(v2 - d * (s - 3))), cols[ch]]
                return mkrdma(src, comm2[ch].at[s - 3], ch, s, tgt2)
            if s < 9:
                src = out_ref.at[prow(qown[ch], m4(v2 + d * (7 - s))), cols[ch]]
                return mkrdma(src, src, ch, s, tgt2)
            src = out_ref.at[qrows(m4(v + d * (10 - s))), cols[ch]]
            return mkrdma(src, src, ch, s, tgt)

        def start_p1(ch, k):
            v, tgt, _, _, d = geom[ch]
            p = p1idx[ch][k]
            src = out_ref.at[prow(m4(v - 2 * d), p), cols[ch]]
            dst = comm1[ch].at[2, pl.ds(p * PROWS, PROWS), :]
            return mkrdma(src, dst, ch, P1SEM[k], tgt)

        def start_p4(ch, k):
            _, tgt, _, _, _ = geom[ch]
            src = out_ref.at[prow(qown[ch], p4idx[ch][k]), cols[ch]]
            return mkrdma(src, src, ch, P4SEM[k], tgt)

        def acc_p1(ch, k):
            p = p1idx[ch][k]
            out_ref[prow(qown[ch], p), cols[ch]] = (
                out_ref[prow(qown[ch], p), cols[ch]]
                + comm1[ch][2, pl.ds(p * PROWS, PROWS), :])

        def accumulate(ch, s):
            v, _, v2, _, d = geom[ch]
            if s < 3:
                rq = m4(v - d * (s + 1))
                out_ref[qrows(rq), cols[ch]] = (
                    out_ref[qrows(rq), cols[ch]] + comm1[ch][s])
            else:
                rp = m4(v2 - d * (s - 2))
                out_ref[prow(qown[ch], rp), cols[ch]] = (
                    out_ref[prow(qown[ch], rp), cols[ch]] + comm2[ch][s - 3])

        def compute_quarter(idx):
            out_ref[qrows(idx), :] = jnp.dot(
                a_ref[qrows(idx), :], b_ref[:, :],
                preferred_element_type=jnp.float32)

        barrier_sem = pltpu.get_barrier_semaphore()
        for nbr in (pr, plq, zr, zl):
            pl.semaphore_signal(barrier_sem, inc=1, device_id=(nbr,),
                                device_id_type=pl.DeviceIdType.MESH)
        pl.semaphore_wait(barrier_sem, 4)

        compute_quarter(q4)
        live = {}
        live["AR"] = start("AR", 0)
        live["AL"] = start("AL", 0)
        pl.when(z4 != q4)(lambda: compute_quarter(z4))
        live["BR"] = start("BR", 0)
        live["BL"] = start("BL", 0)
        for j in range(4):
            pl.when((j != q4) & (j != z4))(lambda j=j: compute_quarter(j))

        p1d = {}
        for s in (0, 1):
            for ch in CHS:
                live[ch].wait_recv()
                accumulate(ch, s)
                if s == 0:
                    live[ch] = start(ch, 1)
                else:
                    p1d[ch] = [start_p1(ch, k) for k in range(4)]

        for ch in CHS:
            p1d[ch][0].wait_recv()
            acc_p1(ch, 0)
            live[ch] = start(ch, 3)
        for t in range(3):
            for ch in CHS:
                p1d[ch][t + 1].wait_recv()
                acc_p1(ch, t + 1)
                live[ch].wait_recv()
                accumulate(ch, 3 + t)
                if t < 2:
                    live[ch] = start(ch, 4 + t)

        p4d = {}
        for ch in CHS:
            live[ch] = start(ch, 6)
            p4d[ch] = [start_p4(ch, 0)]
        for t in range(2):
            for ch in CHS:
                live[ch].wait_recv()
                live[ch] = start(ch, 7 + t)
                p4d[ch].append(start_p4(ch, t + 1))
        for ch in CHS:
            live[ch].wait_recv()
            p4d[ch].append(start_p4(ch, 3))

        for ch in CHS:
            for dsc in p4d[ch]:
                dsc.wait_recv()
            live[ch] = start(ch, 10)
        for ch in CHS:
            live[ch].wait_recv()
            live[ch] = start(ch, 11)
        for ch in CHS:
            live[ch].wait_recv()

        for r in all_rdmas:
            r.wait_send()

    return pl.pallas_call(
        body,
        out_shape=jax.ShapeDtypeStruct((M, N), jnp.float32),
        in_specs=[
            pl.BlockSpec(memory_space=pltpu.VMEM),
            pl.BlockSpec(memory_space=pltpu.VMEM),
        ],
        out_specs=pl.BlockSpec(memory_space=pltpu.VMEM),
        scratch_shapes=(
            [pltpu.VMEM((3, QROWS, STRIP), jnp.float32) for _ in range(4)]
            + [pltpu.VMEM((3, PROWS, STRIP), jnp.float32) for _ in range(4)]
            + [pltpu.SemaphoreType.DMA((NSEM,)) for _ in range(8)]
        ),
        compiler_params=pltpu.CompilerParams(collective_id=0),
    )(A, B)
